# Initial kernel scaffold; baseline (speedup 1.0000x reference)
import jax
import jax.numpy as jnp
from jax.experimental import pallas as pl


def kernel(q, k, v, pt_attn, bias, w1, b1, w2, b2, w_out, b_out):
    raise NotImplementedError("write your pallas kernel here")


# speedup vs baseline: 1.1867x; 1.1867x over previous
import jax
import jax.numpy as jnp
from jax.experimental import pallas as pl
from jax.experimental.pallas import tpu as pltpu

_B_BLK = 4


def _fused_kernel(gate_ref, q_ref, k_ref, v_ref, bias_ref,
                  w_out_ref, b_out_ref, out_ref, weights_ref):
    H = q_ref.shape[1]
    Dk = q_ref.shape[3]
    scale = 1.0 / (Dk ** 0.5)
    w_out = w_out_ref[...]
    b_out = b_out_ref[...]

    for b in range(_B_BLK):
        q = q_ref[b]
        k = k_ref[b]
        v = v_ref[b]
        bias = bias_ref[b]
        gate = gate_ref[b]

        attn = jnp.einsum('hqd,hkd->hqk', q * scale, k,
                          preferred_element_type=jnp.float32) + bias

        m = jnp.max(attn, axis=-1, keepdims=True)
        e = jnp.exp(attn - m)
        denom = jnp.sum(e, axis=-1, keepdims=True)
        w = e * pl.reciprocal(denom, approx=True)
        w = w * gate[:, None, :]
        weights_ref[b] = w

        ctx = jnp.einsum('hqk,hkd->hqd', w, v,
                         preferred_element_type=jnp.float32)
        ctx_cat = jnp.concatenate([ctx[h] for h in range(H)], axis=-1)
        out_ref[b] = jnp.dot(ctx_cat, w_out,
                             preferred_element_type=jnp.float32) + b_out


def kernel(q, k, v, pt_attn, bias, w1, b1, w2, b2, w_out, b_out):
    B, H, Lq, Dk = q.shape
    Lk = k.shape[2]
    Dv = v.shape[3]
    d_model = H * Dv

    hpt = jnp.tanh(jnp.einsum('bhpd,dv->bhpv', pt_attn, w1) + b1)
    gate = jax.nn.sigmoid(jnp.sum(hpt * w2, axis=-1) + b2[0, 0])

    nblk = B // _B_BLK
    graph_out, weights = pl.pallas_call(
        _fused_kernel,
        out_shape=(jax.ShapeDtypeStruct((B, Lq, d_model), jnp.float32),
                   jax.ShapeDtypeStruct((B, H, Lq, Lk), jnp.float32)),
        grid=(nblk,),
        in_specs=[
            pl.BlockSpec((_B_BLK, H, Lk), lambda b: (b, 0, 0)),
            pl.BlockSpec((_B_BLK, H, Lq, Dk), lambda b: (b, 0, 0, 0)),
            pl.BlockSpec((_B_BLK, H, Lk, Dk), lambda b: (b, 0, 0, 0)),
            pl.BlockSpec((_B_BLK, H, Lk, Dv), lambda b: (b, 0, 0, 0)),
            pl.BlockSpec((_B_BLK, H, Lq, Lk), lambda b: (b, 0, 0, 0)),
            pl.BlockSpec((d_model, d_model), lambda b: (0, 0)),
            pl.BlockSpec((1, d_model), lambda b: (0, 0)),
        ],
        out_specs=(pl.BlockSpec((_B_BLK, Lq, d_model), lambda b: (b, 0, 0)),
                   pl.BlockSpec((_B_BLK, H, Lq, Lk), lambda b: (b, 0, 0, 0))),
        compiler_params=pltpu.CompilerParams(
            dimension_semantics=("parallel",),
            vmem_limit_bytes=100 * 1024 * 1024,
        ),
    )(gate, q, k, v, bias, w_out, b_out)

    return graph_out, weights
